# Initial kernel scaffold; baseline (speedup 1.0000x reference)
import jax
import jax.numpy as jnp
from jax.experimental import pallas as pl


def kernel(x, edge_index, batch, W1_l, b1_l, W1_r, W2_l, b2_l, W2_r, gamma, beta, W_out, b_out):
    raise NotImplementedError("write your pallas kernel here")


# speedup vs baseline: 6.6818x; 6.6818x over previous
import functools

import jax
import jax.numpy as jnp
from jax import lax
from jax.experimental import pallas as pl
from jax.experimental.pallas import tpu as pltpu
from jax.experimental.pallas import tpu_sc as plsc

_N = 10000
_E = 320000
_D = 128
_G = 64
_DP = 144
_NSC = 2
_NSUB = 16
_NW = _NSC * _NSUB
_EPW = _E // _NW
_CH = 80
_NCH = _EPW // _CH
_NPAD = 10240
_RPT = _NPAD // _NSUB
_RB = 1000


def _sc_agg_body(table, src, dst, zeros, out, srcb, dstb, rows, acc, sem):
    c = lax.axis_index("c")
    s = lax.axis_index("s")
    wid = c * _NSUB + s

    pltpu.sync_copy(src.at[wid], srcb)
    pltpu.sync_copy(dst.at[wid], dstb)
    pltpu.sync_copy(zeros, acc.at[pl.ds(s * _RPT, _RPT)])
    plsc.subcore_barrier()

    def chunk(j, carry):
        pltpu.async_copy(table.at[srcb.at[j]], rows, sem).wait()
        pltpu.sync_copy(rows, acc.at[dstb.at[j]], add=True)
        return carry

    lax.fori_loop(0, _NCH, chunk, 0)
    plsc.subcore_barrier()
    pltpu.sync_copy(acc.at[pl.ds(s * _RPT, _RPT)],
                    out.at[c, pl.ds(s * _RPT, _RPT)])


@functools.cache
def _sc_agg():
    return pl.kernel(
        _sc_agg_body,
        out_type=jax.ShapeDtypeStruct((_NSC, _NPAD, _DP), jnp.float32),
        mesh=plsc.VectorSubcoreMesh(
            core_axis_name="c", subcore_axis_name="s",
            num_cores=_NSC, num_subcores=_NSUB),
        scratch_types=[
            pltpu.VMEM((_NCH, _CH), jnp.int32),
            pltpu.VMEM((_NCH, _CH), jnp.int32),
            pltpu.VMEM((_CH, _DP), jnp.float32),
            pltpu.VMEM_SHARED((_NPAD, _DP), jnp.float32),
            pltpu.SemaphoreType.DMA,
        ],
        compiler_params=pltpu.CompilerParams(use_tc_tiling_on_sc=False),
    )


def _ones_col(rows):
    col = lax.broadcasted_iota(jnp.int32, (rows, _DP - _D), 1)
    return jnp.where(col == 0, 1.0, 0.0).astype(jnp.float32)


def _pad_body(x_ref, o_ref):
    o_ref[...] = jnp.concatenate([x_ref[...], _ones_col(_RB)], axis=1)


_pad = pl.pallas_call(
    _pad_body,
    grid=(_N // _RB,),
    in_specs=[pl.BlockSpec((_RB, _D), lambda i: (i, 0))],
    out_specs=pl.BlockSpec((_RB, _DP), lambda i: (i, 0)),
    out_shape=jax.ShapeDtypeStruct((_N, _DP), jnp.float32),
)


def _sage_dense(parts, xin, wl, wr, b):
    ssum = parts[0, :, :_D] + parts[1, :, :_D]
    cnt = parts[0, :, _D:_D + 1] + parts[1, :, _D:_D + 1]
    mean = ssum / jnp.maximum(cnt, 1.0)
    dn = (((1,), (1,)), ((), ()))
    acc = lax.dot_general(mean, wl, dn, preferred_element_type=jnp.float32)
    acc = acc + b
    acc = acc + lax.dot_general(xin, wr, dn, preferred_element_type=jnp.float32)
    return jnp.maximum(acc, 0.0)


def _dense_body(parts_ref, x_ref, wl_ref, wr_ref, b_ref, o_ref):
    h = _sage_dense(parts_ref[...], x_ref[...][:, :_D],
                    wl_ref[...], wr_ref[...], b_ref[...])
    o_ref[...] = jnp.concatenate([h, _ones_col(_RB)], axis=1)


_dense1 = pl.pallas_call(
    _dense_body,
    grid=(_N // _RB,),
    in_specs=[
        pl.BlockSpec((_NSC, _RB, _DP), lambda i: (0, i, 0)),
        pl.BlockSpec((_RB, _D), lambda i: (i, 0)),
        pl.BlockSpec((_D, _D), lambda i: (0, 0)),
        pl.BlockSpec((_D, _D), lambda i: (0, 0)),
        pl.BlockSpec((1, _D), lambda i: (0, 0)),
    ],
    out_specs=pl.BlockSpec((_RB, _DP), lambda i: (i, 0)),
    out_shape=jax.ShapeDtypeStruct((_N, _DP), jnp.float32),
)


def _final_body(parts_ref, h_ref, oh_ref, wl_ref, wr_ref, b_ref,
                gamma_ref, beta_ref, wout_ref, bout_ref, o_ref, pooled):
    i = pl.program_id(0)
    h2 = _sage_dense(parts_ref[...], h_ref[...][:, :_D],
                     wl_ref[...], wr_ref[...], b_ref[...])
    contrib = lax.dot_general(oh_ref[...], h2, (((0,), (0,)), ((), ())),
                              preferred_element_type=jnp.float32)

    @pl.when(i == 0)
    def _init():
        pooled[...] = jnp.zeros((_G, _D), jnp.float32)

    pooled[...] += contrib

    @pl.when(i == pl.num_programs(0) - 1)
    def _finish():
        pg = pooled[...]
        mu = jnp.mean(pg, axis=1, keepdims=True)
        var = jnp.mean((pg - mu) ** 2, axis=1, keepdims=True)
        normed = gamma_ref[...] * (pg - mu) * lax.rsqrt(var + 1e-5) \
            + beta_ref[...]
        dn = (((1,), (1,)), ((), ()))
        o_ref[...] = lax.dot_general(
            normed, wout_ref[...], dn,
            preferred_element_type=jnp.float32) + bout_ref[...]


_final = pl.pallas_call(
    _final_body,
    grid=(_N // _RB,),
    in_specs=[
        pl.BlockSpec((_NSC, _RB, _DP), lambda i: (0, i, 0)),
        pl.BlockSpec((_RB, _DP), lambda i: (i, 0)),
        pl.BlockSpec((_RB, _G), lambda i: (i, 0)),
        pl.BlockSpec((_D, _D), lambda i: (0, 0)),
        pl.BlockSpec((_D, _D), lambda i: (0, 0)),
        pl.BlockSpec((1, _D), lambda i: (0, 0)),
        pl.BlockSpec((1, _D), lambda i: (0, 0)),
        pl.BlockSpec((1, _D), lambda i: (0, 0)),
        pl.BlockSpec((_D, _D), lambda i: (0, 0)),
        pl.BlockSpec((1, _D), lambda i: (0, 0)),
    ],
    out_specs=pl.BlockSpec((_G, _D), lambda i: (0, 0)),
    out_shape=jax.ShapeDtypeStruct((_G, _D), jnp.float32),
    scratch_shapes=[pltpu.VMEM((_G, _D), jnp.float32)],
)


@jax.jit
def kernel(x, edge_index, batch, W1_l, b1_l, W1_r, W2_l, b2_l, W2_r,
           gamma, beta, W_out, b_out):
    src3 = edge_index[0].reshape(_NW, _NCH, _CH)
    dst3 = edge_index[1].reshape(_NW, _NCH, _CH)
    zeros = jnp.zeros((_RPT, _DP), jnp.float32)
    oh = (batch[:, None] == jnp.arange(_G, dtype=batch.dtype)[None, :])
    oh = oh.astype(jnp.float32)
    b1 = b1_l.reshape(1, _D)
    b2 = b2_l.reshape(1, _D)
    ga = gamma.reshape(1, _D)
    be = beta.reshape(1, _D)
    bo = b_out.reshape(1, _D)

    agg = _sc_agg()
    xpad = _pad(x)
    parts1 = agg(xpad, src3, dst3, zeros)
    h1pad = _dense1(parts1, x, W1_l, W1_r, b1)
    parts2 = agg(h1pad, src3, dst3, zeros)
    return _final(parts2, h1pad, oh, W2_l, W2_r, b2, ga, be, W_out, bo)
